# baseline (speedup 1.0000x reference)
import functools

import jax
import jax.numpy as jnp
from jax import lax
from jax.experimental import pallas as pl
from jax.experimental.pallas import tpu as pltpu
from jax.experimental.pallas import tpu_sc as plsc

N_NODES = 10000
D_FEAT = 128
N_EDGES = 320000

_NC = 2
_NS = 16
_NW = _NC * _NS
_EPW = N_EDGES // _NW
_B = 80
_CHUNKS = _EPW // _B
_L = 16

_mesh = plsc.VectorSubcoreMesh(core_axis_name="c", subcore_axis_name="s")


@functools.partial(
    pl.kernel,
    out_type=jax.ShapeDtypeStruct((N_EDGES,), jnp.float32),
    mesh=_mesh,
    scratch_types=[
        pltpu.VMEM((_CHUNKS, _B), jnp.int32),
        pltpu.VMEM((_CHUNKS, _B), jnp.int32),
        pltpu.VMEM((2, _B, D_FEAT), jnp.float32),
        pltpu.VMEM((2, _B, D_FEAT), jnp.float32),
        pltpu.VMEM((_EPW,), jnp.float32),
        pltpu.SemaphoreType.DMA((2,)),
    ],
    compiler_params=pltpu.CompilerParams(needs_layout_passes=False),
)
def _gae_sc(z_hbm, src_hbm, dst_hbm, out_hbm,
            src_v, dst_v, rows_s, rows_d, out_v, sems):
    wid = lax.axis_index("s") * _NC + lax.axis_index("c")

    pltpu.sync_copy(src_hbm.at[wid], src_v)
    pltpu.sync_copy(dst_hbm.at[wid], dst_v)

    def _start(c, b):
        pltpu.async_copy(z_hbm.at[src_v.at[c]], rows_s.at[b], sems.at[b])
        pltpu.async_copy(z_hbm.at[dst_v.at[c]], rows_d.at[b], sems.at[b])

    def _wait(c, b):
        pltpu.make_async_copy(z_hbm.at[src_v.at[c]], rows_s.at[b], sems.at[b]).wait()
        pltpu.make_async_copy(z_hbm.at[dst_v.at[c]], rows_d.at[b], sems.at[b]).wait()

    _start(0, 0)

    def chunk_body(c, carry):
        b = c & 1

        @pl.when(c + 1 < _CHUNKS)
        def _():
            _start(c + 1, 1 - b)

        _wait(c, b)

        lane = lax.iota(jnp.int32, _L)

        def group_body(g, carry2):
            row_idx = g * _L + lane

            def d_body(d, carry3):
                accs, cols = carry3
                new_accs = []
                new_cols = []
                for k in range(4):
                    s = plsc.load_gather(rows_s.at[b], [row_idx, cols[k]])
                    t = plsc.load_gather(rows_d.at[b], [row_idx, cols[k]])
                    new_accs.append(accs[k] + s * t)
                    new_cols.append((cols[k] + 4) & (D_FEAT - 1))
                return tuple(new_accs), tuple(new_cols)

            zero = jnp.zeros((_L,), jnp.float32)
            (a0, a1, a2, a3), _ = lax.fori_loop(
                0, D_FEAT // 4, d_body,
                ((zero, zero, zero, zero),
                 (lane, lane + 1, lane + 2, lane + 3)),
                unroll=16)
            dots = (a0 + a1) + (a2 + a3)
            out_v[pl.ds(c * _B + g * _L, _L)] = 1.0 / (1.0 + jnp.exp(-dots))
            return carry2

        lax.fori_loop(0, _B // _L, group_body, 0)
        return carry

    lax.fori_loop(0, _CHUNKS, chunk_body, 0)
    pltpu.sync_copy(out_v, out_hbm.at[pl.ds(wid * _EPW, _EPW)])


def kernel(z, edge_index):
    ei = edge_index.astype(jnp.int32)
    src = ei[0].reshape(_NW, _CHUNKS, _B)
    dst = ei[1].reshape(_NW, _CHUNKS, _B)
    return _gae_sc(z, src, dst)


# speedup vs baseline: 1.0343x; 1.0343x over previous
import functools

import jax
import jax.numpy as jnp
from jax import lax
from jax.experimental import pallas as pl
from jax.experimental.pallas import tpu as pltpu
from jax.experimental.pallas import tpu_sc as plsc

N_NODES = 10000
D_FEAT = 128
N_EDGES = 320000

_NC = 2
_NS = 16
_NW = _NC * _NS
_EPW = N_EDGES // _NW
_B = 80
_CHUNKS = _EPW // _B
_L = 16

_mesh = plsc.VectorSubcoreMesh(core_axis_name="c", subcore_axis_name="s")


@functools.partial(
    pl.kernel,
    out_type=jax.ShapeDtypeStruct((N_EDGES,), jnp.float32),
    mesh=_mesh,
    scratch_types=[
        pltpu.VMEM((_CHUNKS, _B), jnp.int32),
        pltpu.VMEM((_CHUNKS, _B), jnp.int32),
        pltpu.VMEM((2, _B, D_FEAT), jnp.float32),
        pltpu.VMEM((2, _B, D_FEAT), jnp.float32),
        pltpu.VMEM((_EPW,), jnp.float32),
        pltpu.SemaphoreType.DMA((2,)),
    ],
    compiler_params=pltpu.CompilerParams(needs_layout_passes=False),
)
def _gae_sc(z_hbm, src_hbm, dst_hbm, out_hbm,
            src_v, dst_v, rows_s, rows_d, out_v, sems):
    wid = lax.axis_index("s") * _NC + lax.axis_index("c")

    pltpu.sync_copy(src_hbm.at[wid], src_v)
    pltpu.sync_copy(dst_hbm.at[wid], dst_v)

    def _start(c, b):
        pltpu.async_copy(z_hbm.at[src_v.at[c]], rows_s.at[b], sems.at[b])
        pltpu.async_copy(z_hbm.at[dst_v.at[c]], rows_d.at[b], sems.at[b])

    def _wait(c, b):
        pltpu.make_async_copy(z_hbm.at[src_v.at[c]], rows_s.at[b], sems.at[b]).wait()
        pltpu.make_async_copy(z_hbm.at[dst_v.at[c]], rows_d.at[b], sems.at[b]).wait()

    _start(0, 0)

    def chunk_body(c, carry):
        b = c & 1

        @pl.when(c + 1 < _CHUNKS)
        def _():
            _start(c + 1, 1 - b)

        _wait(c, b)

        lane = lax.iota(jnp.int32, _L)

        def group_body(g, carry2):
            row_idx = g * _L + lane

            _K = 8

            def d_body(d, carry3):
                accs, cols = carry3
                new_accs = []
                new_cols = []
                for k in range(_K):
                    s = plsc.load_gather(rows_s.at[b], [row_idx, cols[k]])
                    t = plsc.load_gather(rows_d.at[b], [row_idx, cols[k]])
                    new_accs.append(accs[k] + s * t)
                    new_cols.append((cols[k] + _K) & (D_FEAT - 1))
                return tuple(new_accs), tuple(new_cols)

            zero = jnp.zeros((_L,), jnp.float32)
            accs, _ = lax.fori_loop(
                0, D_FEAT // _K, d_body,
                ((zero,) * _K,
                 tuple(lane + k for k in range(_K))),
                unroll=4)
            while len(accs) > 1:
                accs = tuple(accs[i] + accs[i + 1]
                             for i in range(0, len(accs), 2))
            dots = accs[0]
            out_v[pl.ds(c * _B + g * _L, _L)] = 1.0 / (1.0 + jnp.exp(-dots))
            return carry2

        lax.fori_loop(0, _B // _L, group_body, 0)
        return carry

    lax.fori_loop(0, _CHUNKS, chunk_body, 0)
    pltpu.sync_copy(out_v, out_hbm.at[pl.ds(wid * _EPW, _EPW)])


def kernel(z, edge_index):
    ei = edge_index.astype(jnp.int32)
    src = ei[0].reshape(_NW, _CHUNKS, _B)
    dst = ei[1].reshape(_NW, _CHUNKS, _B)
    return _gae_sc(z, src, dst)
